# baseline (speedup 1.0000x reference)
import functools

import jax
import jax.numpy as jnp
from jax import lax
from jax.experimental import pallas as pl
from jax.experimental.pallas import tpu as pltpu
from jax.experimental.pallas import tpu_sc as plsc

_NC = 2
_NS = 16
_L = 16
_NW = _NC * _NS
_K = 128
_NBUF = 4


@functools.lru_cache(maxsize=None)
def _make_spmm(rows, w, nch):
    epw = nch * _K
    rpt = rows // _NS
    nchq = nch // _NBUF
    dump = rows - 1
    shift = max(1, (2 * rows - 1).bit_length())
    mask = (1 << shift) - 1
    mesh = plsc.VectorSubcoreMesh(core_axis_name="c", subcore_axis_name="s")

    @functools.partial(
        pl.kernel,
        mesh=mesh,
        out_type=jax.ShapeDtypeStruct((_NC, rows, w), jnp.float32),
        scratch_types=[
            pltpu.VMEM((nch, _K), jnp.int32),
            pltpu.VMEM((epw,), jnp.float32),
            pltpu.VMEM((_NBUF, _K), jnp.int32),
            pltpu.VMEM((_NBUF, _K), jnp.int32),
            pltpu.VMEM((_NBUF, _K, w), jnp.float32),
            pltpu.VMEM_SHARED((rows, w), jnp.float32),
        ]
        + [pltpu.SemaphoreType.DMA] * (2 * _NBUF),
        compiler_params=pltpu.CompilerParams(use_tc_tiling_on_sc=False),
    )
    def spmm(s_hbm, packed_hbm, ew_hbm, out_hbm,
             packed_v, ew_v, sidx, didx, rows_v, acc_sh, *sems):
        gsem = sems[:_NBUF]
        ssem = sems[_NBUF:]
        c = lax.axis_index("c")
        s = lax.axis_index("s")
        wid = c * _NS + s
        row0 = c * rows

        pltpu.sync_copy(packed_hbm.at[wid], packed_v)
        pltpu.sync_copy(ew_hbm.at[wid], ew_v)

        def zbody(i, carry):
            for f in range(w // _L):
                rows_v[0, i, pl.ds(f * _L, _L)] = jnp.zeros((_L,), jnp.float32)
            return carry

        lax.fori_loop(0, _K, zbody, 0)
        for q in range(rpt // _K):
            pltpu.sync_copy(rows_v.at[0],
                            acc_sh.at[pl.ds(s * rpt + q * _K, _K)])
        rem = rpt % _K
        if rem:
            pltpu.sync_copy(
                rows_v.at[0].at[pl.ds(0, rem)],
                acc_sh.at[pl.ds(s * rpt + (rpt // _K) * _K, rem)])
        plsc.subcore_barrier()

        def decode(j, r):
            for g in range(_K // _L):
                v = packed_v[j, pl.ds(g * _L, _L)]
                sidx[r, pl.ds(g * _L, _L)] = jnp.bitwise_and(v, mask)
                d_local = lax.shift_right_logical(v, shift) - row0
                ok = jnp.logical_and(d_local >= 0, d_local < rows)
                didx[r, pl.ds(g * _L, _L)] = jnp.where(
                    ok, d_local, jnp.full((_L,), dump, jnp.int32))

        def gather_start(j, r):
            decode(j, r)
            pltpu.async_copy(s_hbm.at[sidx.at[r]], rows_v.at[r], gsem[r])

        def gather_wait(j, r):
            pltpu.make_async_copy(
                s_hbm.at[sidx.at[r]], rows_v.at[r], gsem[r]).wait()

        def scatter_start(j, r):
            pltpu.async_copy(
                rows_v.at[r], acc_sh.at[didx.at[r]], ssem[r], add=True)

        def scatter_wait(j, r):
            pltpu.make_async_copy(
                rows_v.at[r], acc_sh.at[didx.at[r]], ssem[r]).wait()

        def scale(j, r):
            base = j * _K

            def g_body(g, carry):
                ew_g = ew_v[pl.ds(base + g * _L, _L)]
                for e in range(_L):
                    ewb = lax.gather(
                        ew_g, jnp.full((_L, 1), e, jnp.int32),
                        lax.GatherDimensionNumbers(
                            offset_dims=(), collapsed_slice_dims=(0,),
                            start_index_map=(0,)),
                        slice_sizes=(1,),
                        mode=lax.GatherScatterMode.PROMISE_IN_BOUNDS)
                    row = g * _L + e
                    for f in range(w // _L):
                        cur = rows_v[r, row, pl.ds(f * _L, _L)]
                        rows_v[r, row, pl.ds(f * _L, _L)] = cur * ewb
                return carry

            lax.fori_loop(0, _K // _L, g_body, 0)

        def step(j, r, do_swait, do_gstart):
            gather_wait(j, r)
            scale(j, r)
            scatter_start(j, r)
            r3 = (r + _NBUF - 1) % _NBUF
            if do_swait:
                scatter_wait(j - 1, r3)
            if do_gstart:
                gather_start(j + _NBUF - 1, r3)

        for j in range(_NBUF - 1):
            gather_start(j, j)
        for r in range(_NBUF):
            step(r, r, do_swait=(r >= 1), do_gstart=True)

        def middle(jq, carry):
            for r in range(_NBUF):
                step(jq * _NBUF + r, r, do_swait=True, do_gstart=True)
            return carry

        lax.fori_loop(1, nchq - 1, middle, 0)

        jlast = (nchq - 1) * _NBUF
        for r in range(_NBUF):
            step(jlast + r, r, do_swait=(r == 0), do_gstart=(r == 0))
        for m in range(_NBUF):
            scatter_wait(jlast + m, m)

        plsc.subcore_barrier()
        pltpu.sync_copy(acc_sh.at[pl.ds(s * rpt, rpt)],
                        out_hbm.at[c].at[pl.ds(s * rpt, rpt)])

    return spmm


def _mm_plain(x, w):
    n, f = x.shape
    bm = n // 8

    def kfn(x_ref, w_ref, o_ref):
        o_ref[...] = jnp.dot(x_ref[...], w_ref[...],
                             preferred_element_type=jnp.float32)

    return pl.pallas_call(
        kfn,
        grid=(n // bm,),
        in_specs=[pl.BlockSpec((bm, f), lambda i: (i, 0)),
                  pl.BlockSpec(w.shape, lambda i: (0, 0))],
        out_specs=pl.BlockSpec((bm, w.shape[1]), lambda i: (i, 0)),
        out_shape=jax.ShapeDtypeStruct((n, w.shape[1]), jnp.float32),
    )(x, w)


def _mm_fused(p, b, w):
    n, h = p.shape
    bm = n // 8
    b2 = b.reshape(1, h)

    def kfn(p_ref, b_ref, w_ref, o_ref):
        hid = jnp.maximum(p_ref[...] + b_ref[...], 0.0)
        o_ref[...] = jnp.dot(hid, w_ref[...],
                             preferred_element_type=jnp.float32)

    return pl.pallas_call(
        kfn,
        grid=(n // bm,),
        in_specs=[pl.BlockSpec((bm, h), lambda i: (i, 0)),
                  pl.BlockSpec((1, h), lambda i: (0, 0)),
                  pl.BlockSpec(w.shape, lambda i: (0, 0))],
        out_specs=pl.BlockSpec((bm, w.shape[1]), lambda i: (i, 0)),
        out_shape=jax.ShapeDtypeStruct((n, w.shape[1]), jnp.float32),
    )(p, b2, w)


def _bias_add(p, b):
    n, cdim = p.shape
    bm = n // 8
    b2 = b.reshape(1, cdim)

    def kfn(p_ref, b_ref, o_ref):
        o_ref[...] = p_ref[...] + b_ref[...]

    return pl.pallas_call(
        kfn,
        grid=(n // bm,),
        in_specs=[pl.BlockSpec((bm, cdim), lambda i: (i, 0)),
                  pl.BlockSpec((1, cdim), lambda i: (0, 0))],
        out_specs=pl.BlockSpec((bm, cdim), lambda i: (i, 0)),
        out_shape=jax.ShapeDtypeStruct((n, cdim), jnp.float32),
    )(p, b2)


def kernel(x, edge_index, edge_weight, W1, b1, W2, b2, W3, b3):
    n = x.shape[0]
    e = edge_index.shape[1]
    feat = x.shape[1]
    cdim = W3.shape[1]

    split = n // 2
    half = max(split, n - split)
    rows = -(-(half + 1) // _K) * _K
    off = rows - split
    shift = max(1, (2 * rows - 1).bit_length())

    per_w = -(-e // (_NW * _K * _NBUF)) * _K * _NBUF
    nch = per_w // _K
    pad = _NW * per_w - e
    dst = jnp.concatenate([edge_index[0], jnp.zeros((pad,), jnp.int32)])
    src = jnp.concatenate([edge_index[1], jnp.zeros((pad,), jnp.int32)])
    ew = jnp.concatenate([edge_weight, jnp.zeros((pad,), jnp.float32)])
    srcr = src + jnp.where(src >= split, off, 0)
    dstr = dst + jnp.where(dst >= split, off, 0)
    packed = jnp.bitwise_or(jnp.left_shift(dstr, shift), srcr)
    pk = packed.reshape(_NW, nch, _K)
    ew2 = ew.reshape(_NW, per_w)

    xp = jnp.concatenate([
        x[:split], jnp.zeros((rows - split, feat), jnp.float32),
        x[split:], jnp.zeros((rows - (n - split), feat), jnp.float32)])

    spmm_w = _make_spmm(rows, W1.shape[1], nch)
    spmm_c = _make_spmm(rows, cdim, nch)

    s1 = _mm_plain(xp, W1)
    t1 = spmm_w(s1, pk, ew2).reshape(2 * rows, W1.shape[1])
    s2 = _mm_fused(t1, b1, W2)
    t2 = spmm_w(s2, pk, ew2).reshape(2 * rows, W2.shape[1])
    s3 = _mm_fused(t2, b2, W3)
    t3 = spmm_c(s3, pk, ew2).reshape(2 * rows, cdim)
    y = _bias_add(t3, b3)
    return jnp.concatenate([y[:split], y[rows:rows + (n - split)]])


# speedup vs baseline: 1.2184x; 1.2184x over previous
import functools

import jax
import jax.numpy as jnp
from jax import lax
from jax.experimental import pallas as pl
from jax.experimental.pallas import tpu as pltpu
from jax.experimental.pallas import tpu_sc as plsc

_NC = 2
_NS = 16
_L = 16
_NW = _NC * _NS
_K = 128
_NBUF = 4

@functools.lru_cache(maxsize=None)
def _make_spmm(n, w, nch, feature_split):
    epw = nch * _K
    rpt = n // _NS
    nchq = nch // _NBUF
    shift = max(1, (n - 1).bit_length())
    mask = (1 << shift) - 1
    mesh = plsc.VectorSubcoreMesh(core_axis_name="c", subcore_axis_name="s")

    gather_dtype = jnp.float32
    scratch = [
        pltpu.VMEM((nch, _K), jnp.int32),
        pltpu.VMEM((epw,), jnp.float32),
        pltpu.VMEM((_NBUF, _K), jnp.int32),
        pltpu.VMEM((_NBUF, _K), jnp.int32),
        pltpu.VMEM((_NBUF, _K, w), gather_dtype),
        pltpu.VMEM_SHARED((n, w), jnp.float32),
    ] + [pltpu.SemaphoreType.DMA] * (2 * _NBUF)

    @functools.partial(
        pl.kernel,
        mesh=mesh,
        out_type=jax.ShapeDtypeStruct((_NC, n, w), jnp.float32),
        scratch_types=scratch,
        compiler_params=pltpu.CompilerParams(use_tc_tiling_on_sc=False),
    )
    def spmm(s_hbm, packed_hbm, ew_hbm, out_hbm,
             packed_v, ew_v, sidx, didx, rows_v, acc_sh, *sems):
        gsem = sems[:_NBUF]
        ssem = sems[_NBUF:]
        c = lax.axis_index("c")
        s = lax.axis_index("s")
        if feature_split:
            edge_slot = s
            table = s_hbm.at[c]
        else:
            edge_slot = c * _NS + s
            table = s_hbm

        pltpu.sync_copy(packed_hbm.at[edge_slot], packed_v)
        pltpu.sync_copy(ew_hbm.at[edge_slot], ew_v)

        def zbody(i, carry):
            for f in range(w // _L):
                rows_v[0, i, pl.ds(f * _L, _L)] = jnp.zeros((_L,), jnp.float32)
            return carry

        lax.fori_loop(0, _K, zbody, 0)
        for q in range(rpt // _K):
            pltpu.sync_copy(rows_v.at[0],
                            acc_sh.at[pl.ds(s * rpt + q * _K, _K)])
        rem = rpt % _K
        if rem:
            pltpu.sync_copy(
                rows_v.at[0].at[pl.ds(0, rem)],
                acc_sh.at[pl.ds(s * rpt + (rpt // _K) * _K, rem)])
        plsc.subcore_barrier()

        def decode(j, r):
            for g in range(_K // _L):
                v = packed_v[j, pl.ds(g * _L, _L)]
                sidx[r, pl.ds(g * _L, _L)] = jnp.bitwise_and(v, mask)
                didx[r, pl.ds(g * _L, _L)] = lax.shift_right_logical(v, shift)

        def gather_start(j, r):
            decode(j, r)
            pltpu.async_copy(table.at[sidx.at[r]], rows_v.at[r], gsem[r])

        def gather_wait(j, r):
            pltpu.make_async_copy(
                table.at[sidx.at[r]], rows_v.at[r], gsem[r]).wait()

        def scatter_start(j, r):
            pltpu.async_copy(
                rows_v.at[r], acc_sh.at[didx.at[r]], ssem[r], add=True)

        def scatter_wait(j, r):
            pltpu.make_async_copy(
                rows_v.at[r], acc_sh.at[didx.at[r]], ssem[r]).wait()

        def scale(j, r):
            base = j * _K

            def g_body(g, carry):
                ew_g = ew_v[pl.ds(base + g * _L, _L)]
                for e in range(_L):
                    ewb = lax.gather(
                        ew_g, jnp.full((_L, 1), e, jnp.int32),
                        lax.GatherDimensionNumbers(
                            offset_dims=(), collapsed_slice_dims=(0,),
                            start_index_map=(0,)),
                        slice_sizes=(1,),
                        mode=lax.GatherScatterMode.PROMISE_IN_BOUNDS)
                    row = g * _L + e
                    for f in range(w // _L):
                        cur = rows_v[r, row, pl.ds(f * _L, _L)]
                        rows_v[r, row, pl.ds(f * _L, _L)] = cur * ewb
                return carry

            lax.fori_loop(0, _K // _L, g_body, 0)

        def step(j, r, do_swait, do_gstart):
            gather_wait(j, r)
            scale(j, r)
            scatter_start(j, r)
            r3 = (r + _NBUF - 1) % _NBUF
            if do_swait:
                scatter_wait(j - 1, r3)
            if do_gstart:
                gather_start(j + _NBUF - 1, r3)

        for j in range(_NBUF - 1):
            gather_start(j, j)
        for r in range(_NBUF):
            step(r, r, do_swait=(r >= 1), do_gstart=True)

        def middle(jq, carry):
            for r in range(_NBUF):
                step(jq * _NBUF + r, r, do_swait=True, do_gstart=True)
            return carry

        lax.fori_loop(1, nchq - 1, middle, 0)

        jlast = (nchq - 1) * _NBUF
        for r in range(_NBUF):
            step(jlast + r, r, do_swait=(r == 0), do_gstart=(r == 0))
        for m in range(_NBUF):
            scatter_wait(jlast + m, m)

        plsc.subcore_barrier()
        pltpu.sync_copy(acc_sh.at[pl.ds(s * rpt, rpt)],
                        out_hbm.at[c].at[pl.ds(s * rpt, rpt)])

    return spmm


def _mm_split(x, w):
    n, f = x.shape
    bm = n // 8
    wh = w.shape[1] // 2
    wa, wb = w[:, :wh], w[:, wh:]

    def kfn(x_ref, wa_ref, wb_ref, o_ref):
        xv = x_ref[...]
        o_ref[0] = jnp.dot(xv, wa_ref[...],
                           preferred_element_type=jnp.float32)
        o_ref[1] = jnp.dot(xv, wb_ref[...],
                           preferred_element_type=jnp.float32)

    return pl.pallas_call(
        kfn,
        grid=(n // bm,),
        in_specs=[pl.BlockSpec((bm, f), lambda i: (i, 0)),
                  pl.BlockSpec((f, wh), lambda i: (0, 0)),
                  pl.BlockSpec((f, wh), lambda i: (0, 0))],
        out_specs=pl.BlockSpec((2, bm, wh), lambda i: (0, i, 0)),
        out_shape=jax.ShapeDtypeStruct((2, n, wh), jnp.float32),
    )(x, wa, wb)


def _mm_fused_split(p, b, w):
    _, n, ph = p.shape
    h = 2 * ph
    bm = n // 8
    wh = w.shape[1] // 2
    wa, wb = w[:, :wh], w[:, wh:]
    b2 = b.reshape(1, h)

    def kfn(p_ref, b_ref, wa_ref, wb_ref, o_ref):
        hid = jnp.concatenate([p_ref[0], p_ref[1]], axis=1) + b_ref[...]
        hid = jnp.maximum(hid, 0.0)
        o_ref[0] = jnp.dot(hid, wa_ref[...],
                           preferred_element_type=jnp.float32)
        o_ref[1] = jnp.dot(hid, wb_ref[...],
                           preferred_element_type=jnp.float32)

    return pl.pallas_call(
        kfn,
        grid=(n // bm,),
        in_specs=[pl.BlockSpec((2, bm, ph), lambda i: (0, i, 0)),
                  pl.BlockSpec((1, h), lambda i: (0, 0)),
                  pl.BlockSpec((w.shape[0], wh), lambda i: (0, 0)),
                  pl.BlockSpec((w.shape[0], wh), lambda i: (0, 0))],
        out_specs=pl.BlockSpec((2, bm, wh), lambda i: (0, i, 0)),
        out_shape=jax.ShapeDtypeStruct((2, n, wh), jnp.float32),
    )(p, b2, wa, wb)


def _mm_fused_narrow(p, b, w):
    _, n, ph = p.shape
    h = 2 * ph
    bm = n // 8
    b2 = b.reshape(1, h)

    def kfn(p_ref, b_ref, w_ref, o_ref):
        hid = jnp.concatenate([p_ref[0], p_ref[1]], axis=1) + b_ref[...]
        hid = jnp.maximum(hid, 0.0)
        o_ref[...] = jnp.dot(hid, w_ref[...],
                             preferred_element_type=jnp.float32)

    return pl.pallas_call(
        kfn,
        grid=(n // bm,),
        in_specs=[pl.BlockSpec((2, bm, ph), lambda i: (0, i, 0)),
                  pl.BlockSpec((1, h), lambda i: (0, 0)),
                  pl.BlockSpec(w.shape, lambda i: (0, 0))],
        out_specs=pl.BlockSpec((bm, w.shape[1]), lambda i: (i, 0)),
        out_shape=jax.ShapeDtypeStruct((n, w.shape[1]), jnp.float32),
    )(p, b2, w)


def _final_add(p, b):
    _, n, cdim = p.shape
    bm = n // 8
    b2 = b.reshape(1, cdim)

    def kfn(p_ref, b_ref, o_ref):
        o_ref[...] = p_ref[0] + p_ref[1] + b_ref[...]

    return pl.pallas_call(
        kfn,
        grid=(n // bm,),
        in_specs=[pl.BlockSpec((2, bm, cdim), lambda i: (0, i, 0)),
                  pl.BlockSpec((1, cdim), lambda i: (0, 0))],
        out_specs=pl.BlockSpec((bm, cdim), lambda i: (i, 0)),
        out_shape=jax.ShapeDtypeStruct((n, cdim), jnp.float32),
    )(p, b2)


def kernel(x, edge_index, edge_weight, W1, b1, W2, b2, W3, b3):
    n = x.shape[0]
    e = edge_index.shape[1]
    cdim = W3.shape[1]
    hdim = W1.shape[1]
    npad = -(-n // _K) * _K
    shift = max(1, (npad - 1).bit_length())

    per16 = -(-e // (_NS * _K * _NBUF * 2)) * _K * _NBUF * 2
    nch16 = per16 // _K
    pad = _NS * per16 - e
    dst = jnp.concatenate([edge_index[0], jnp.zeros((pad,), jnp.int32)])
    src = jnp.concatenate([edge_index[1], jnp.zeros((pad,), jnp.int32)])
    ew = jnp.concatenate([edge_weight, jnp.zeros((pad,), jnp.float32)])
    packed = jnp.bitwise_or(jnp.left_shift(dst, shift), src)
    pk16 = packed.reshape(_NS, nch16, _K)
    ew16 = ew.reshape(_NS, per16)
    nch32 = nch16 // 2
    pk32 = packed.reshape(_NW, nch32, _K)
    ew32 = ew.reshape(_NW, per16 // 2)

    xp = jnp.concatenate([x, jnp.zeros((npad - n, x.shape[1]), jnp.float32)])

    spmm_wide = _make_spmm(npad, hdim // 2, nch16, True)
    spmm_narrow = _make_spmm(npad, cdim, nch32, False)

    s1 = _mm_split(xp, W1)
    p1 = spmm_wide(s1, pk16, ew16)
    s2 = _mm_fused_split(p1, b1, W2)
    p2 = spmm_wide(s2, pk16, ew16)
    s3 = _mm_fused_narrow(p2, b2, W3)
    p3 = spmm_narrow(s3, pk32, ew32)
    return _final_add(p3, b3)[:n]


# speedup vs baseline: 1.2627x; 1.0363x over previous
import functools

import jax
import jax.numpy as jnp
from jax import lax
from jax.experimental import pallas as pl
from jax.experimental.pallas import tpu as pltpu
from jax.experimental.pallas import tpu_sc as plsc

_NC = 2
_NS = 16
_L = 16
_NW = _NC * _NS
_K = 128
_NBUF = 5

@functools.lru_cache(maxsize=None)
def _make_spmm(n, w, nch, feature_split):
    epw = nch * _K
    rpt = n // _NS
    nchq = nch // _NBUF
    shift = max(1, (n - 1).bit_length())
    mask = (1 << shift) - 1
    mesh = plsc.VectorSubcoreMesh(core_axis_name="c", subcore_axis_name="s")

    gather_dtype = jnp.float32
    scratch = [
        pltpu.VMEM((nch, _K), jnp.int32),
        pltpu.VMEM((epw,), jnp.float32),
        pltpu.VMEM((_NBUF, _K), jnp.int32),
        pltpu.VMEM((_NBUF, _K), jnp.int32),
        pltpu.VMEM((_NBUF, _K, w), gather_dtype),
        pltpu.VMEM_SHARED((n, w), jnp.float32),
    ] + [pltpu.SemaphoreType.DMA] * (2 * _NBUF)

    @functools.partial(
        pl.kernel,
        mesh=mesh,
        out_type=jax.ShapeDtypeStruct((_NC, n, w), jnp.float32),
        scratch_types=scratch,
        compiler_params=pltpu.CompilerParams(use_tc_tiling_on_sc=False),
    )
    def spmm(s_hbm, packed_hbm, ew_hbm, out_hbm,
             packed_v, ew_v, sidx, didx, rows_v, acc_sh, *sems):
        gsem = sems[:_NBUF]
        ssem = sems[_NBUF:]
        c = lax.axis_index("c")
        s = lax.axis_index("s")
        if feature_split:
            edge_slot = s
            table = s_hbm.at[c]
        else:
            edge_slot = c * _NS + s
            table = s_hbm

        pltpu.sync_copy(packed_hbm.at[edge_slot], packed_v)
        pltpu.sync_copy(ew_hbm.at[edge_slot], ew_v)

        def zbody(i, carry):
            for f in range(w // _L):
                rows_v[0, i, pl.ds(f * _L, _L)] = jnp.zeros((_L,), jnp.float32)
            return carry

        lax.fori_loop(0, _K, zbody, 0)
        for q in range(rpt // _K):
            pltpu.sync_copy(rows_v.at[0],
                            acc_sh.at[pl.ds(s * rpt + q * _K, _K)])
        rem = rpt % _K
        if rem:
            pltpu.sync_copy(
                rows_v.at[0].at[pl.ds(0, rem)],
                acc_sh.at[pl.ds(s * rpt + (rpt // _K) * _K, rem)])
        plsc.subcore_barrier()

        def decode(j, r):
            for g in range(_K // _L):
                v = packed_v[j, pl.ds(g * _L, _L)]
                sidx[r, pl.ds(g * _L, _L)] = jnp.bitwise_and(v, mask)
                didx[r, pl.ds(g * _L, _L)] = lax.shift_right_logical(v, shift)

        def gather_start(j, r):
            decode(j, r)
            pltpu.async_copy(table.at[sidx.at[r]], rows_v.at[r], gsem[r])

        def gather_wait(j, r):
            pltpu.make_async_copy(
                table.at[sidx.at[r]], rows_v.at[r], gsem[r]).wait()

        def scatter_start(j, r):
            pltpu.async_copy(
                rows_v.at[r], acc_sh.at[didx.at[r]], ssem[r], add=True)

        def scatter_wait(j, r):
            pltpu.make_async_copy(
                rows_v.at[r], acc_sh.at[didx.at[r]], ssem[r]).wait()

        def scale(j, r):
            base = j * _K

            def g_body(g, carry):
                ew_g = ew_v[pl.ds(base + g * _L, _L)]
                for e in range(_L):
                    ewb = lax.gather(
                        ew_g, jnp.full((_L, 1), e, jnp.int32),
                        lax.GatherDimensionNumbers(
                            offset_dims=(), collapsed_slice_dims=(0,),
                            start_index_map=(0,)),
                        slice_sizes=(1,),
                        mode=lax.GatherScatterMode.PROMISE_IN_BOUNDS)
                    row = g * _L + e
                    for f in range(w // _L):
                        cur = rows_v[r, row, pl.ds(f * _L, _L)]
                        rows_v[r, row, pl.ds(f * _L, _L)] = cur * ewb
                return carry

            lax.fori_loop(0, _K // _L, g_body, 0)

        def step(j, r, do_swait, do_gstart):
            gather_wait(j, r)
            scale(j, r)
            scatter_start(j, r)
            r3 = (r + _NBUF - 1) % _NBUF
            if do_swait:
                scatter_wait(j - 1, r3)
            if do_gstart:
                gather_start(j + _NBUF - 1, r3)

        for j in range(_NBUF - 1):
            gather_start(j, j)
        for r in range(_NBUF):
            step(r, r, do_swait=(r >= 1), do_gstart=True)

        def middle(jq, carry):
            for r in range(_NBUF):
                step(jq * _NBUF + r, r, do_swait=True, do_gstart=True)
            return carry

        lax.fori_loop(1, nchq - 1, middle, 0)

        jlast = (nchq - 1) * _NBUF
        for r in range(_NBUF):
            step(jlast + r, r, do_swait=(r == 0), do_gstart=(r == 0))
        for m in range(_NBUF):
            scatter_wait(jlast + m, m)

        plsc.subcore_barrier()
        pltpu.sync_copy(acc_sh.at[pl.ds(s * rpt, rpt)],
                        out_hbm.at[c].at[pl.ds(s * rpt, rpt)])

    return spmm


def _mm_split(x, w):
    n, f = x.shape
    bm = n // 8
    wh = w.shape[1] // 2
    wa, wb = w[:, :wh], w[:, wh:]

    def kfn(x_ref, wa_ref, wb_ref, o_ref):
        xv = x_ref[...]
        o_ref[0] = jnp.dot(xv, wa_ref[...],
                           preferred_element_type=jnp.float32)
        o_ref[1] = jnp.dot(xv, wb_ref[...],
                           preferred_element_type=jnp.float32)

    return pl.pallas_call(
        kfn,
        grid=(n // bm,),
        in_specs=[pl.BlockSpec((bm, f), lambda i: (i, 0)),
                  pl.BlockSpec((f, wh), lambda i: (0, 0)),
                  pl.BlockSpec((f, wh), lambda i: (0, 0))],
        out_specs=pl.BlockSpec((2, bm, wh), lambda i: (0, i, 0)),
        out_shape=jax.ShapeDtypeStruct((2, n, wh), jnp.float32),
    )(x, wa, wb)


def _mm_fused_split(p, b, w):
    _, n, ph = p.shape
    h = 2 * ph
    bm = n // 8
    wh = w.shape[1] // 2
    wa, wb = w[:, :wh], w[:, wh:]
    b2 = b.reshape(1, h)

    def kfn(p_ref, b_ref, wa_ref, wb_ref, o_ref):
        hid = jnp.concatenate([p_ref[0], p_ref[1]], axis=1) + b_ref[...]
        hid = jnp.maximum(hid, 0.0)
        o_ref[0] = jnp.dot(hid, wa_ref[...],
                           preferred_element_type=jnp.float32)
        o_ref[1] = jnp.dot(hid, wb_ref[...],
                           preferred_element_type=jnp.float32)

    return pl.pallas_call(
        kfn,
        grid=(n // bm,),
        in_specs=[pl.BlockSpec((2, bm, ph), lambda i: (0, i, 0)),
                  pl.BlockSpec((1, h), lambda i: (0, 0)),
                  pl.BlockSpec((w.shape[0], wh), lambda i: (0, 0)),
                  pl.BlockSpec((w.shape[0], wh), lambda i: (0, 0))],
        out_specs=pl.BlockSpec((2, bm, wh), lambda i: (0, i, 0)),
        out_shape=jax.ShapeDtypeStruct((2, n, wh), jnp.float32),
    )(p, b2, wa, wb)


def _mm_fused_narrow(p, b, w):
    _, n, ph = p.shape
    h = 2 * ph
    bm = n // 8
    b2 = b.reshape(1, h)

    def kfn(p_ref, b_ref, w_ref, o_ref):
        hid = jnp.concatenate([p_ref[0], p_ref[1]], axis=1) + b_ref[...]
        hid = jnp.maximum(hid, 0.0)
        o_ref[...] = jnp.dot(hid, w_ref[...],
                             preferred_element_type=jnp.float32)

    return pl.pallas_call(
        kfn,
        grid=(n // bm,),
        in_specs=[pl.BlockSpec((2, bm, ph), lambda i: (0, i, 0)),
                  pl.BlockSpec((1, h), lambda i: (0, 0)),
                  pl.BlockSpec(w.shape, lambda i: (0, 0))],
        out_specs=pl.BlockSpec((bm, w.shape[1]), lambda i: (i, 0)),
        out_shape=jax.ShapeDtypeStruct((n, w.shape[1]), jnp.float32),
    )(p, b2, w)


def _final_add(p, b):
    _, n, cdim = p.shape
    bm = n // 8
    b2 = b.reshape(1, cdim)

    def kfn(p_ref, b_ref, o_ref):
        o_ref[...] = p_ref[0] + p_ref[1] + b_ref[...]

    return pl.pallas_call(
        kfn,
        grid=(n // bm,),
        in_specs=[pl.BlockSpec((2, bm, cdim), lambda i: (0, i, 0)),
                  pl.BlockSpec((1, cdim), lambda i: (0, 0))],
        out_specs=pl.BlockSpec((bm, cdim), lambda i: (i, 0)),
        out_shape=jax.ShapeDtypeStruct((n, cdim), jnp.float32),
    )(p, b2)


def kernel(x, edge_index, edge_weight, W1, b1, W2, b2, W3, b3):
    n = x.shape[0]
    e = edge_index.shape[1]
    cdim = W3.shape[1]
    hdim = W1.shape[1]
    npad = -(-n // _K) * _K
    shift = max(1, (npad - 1).bit_length())

    per16 = -(-e // (_NS * _K * _NBUF * 2)) * _K * _NBUF * 2
    nch16 = per16 // _K
    pad = _NS * per16 - e
    dst = jnp.concatenate([edge_index[0], jnp.zeros((pad,), jnp.int32)])
    src = jnp.concatenate([edge_index[1], jnp.zeros((pad,), jnp.int32)])
    ew = jnp.concatenate([edge_weight, jnp.zeros((pad,), jnp.float32)])
    packed = jnp.bitwise_or(jnp.left_shift(dst, shift), src)
    pk16 = packed.reshape(_NS, nch16, _K)
    ew16 = ew.reshape(_NS, per16)
    nch32 = nch16 // 2
    pk32 = packed.reshape(_NW, nch32, _K)
    ew32 = ew.reshape(_NW, per16 // 2)

    xp = jnp.concatenate([x, jnp.zeros((npad - n, x.shape[1]), jnp.float32)])

    spmm_wide = _make_spmm(npad, hdim // 2, nch16, True)
    spmm_narrow = _make_spmm(npad, cdim, nch32, False)

    s1 = _mm_split(xp, W1)
    p1 = spmm_wide(s1, pk16, ew16)
    s2 = _mm_fused_split(p1, b1, W2)
    p2 = spmm_wide(s2, pk16, ew16)
    s3 = _mm_fused_narrow(p2, b2, W3)
    p3 = spmm_narrow(s3, pk32, ew32)
    return _final_add(p3, b3)[:n]
